# Initial kernel scaffold; baseline (speedup 1.0000x reference)
import jax
import jax.numpy as jnp
from jax.experimental import pallas as pl


def kernel(tokens, W_E):
    raise NotImplementedError("write your pallas kernel here")


# speedup vs baseline: 1.7678x; 1.7678x over previous
import functools
import jax
import jax.numpy as jnp
from jax import lax
from jax.experimental import pallas as pl
from jax.experimental.pallas import tpu as pltpu
from jax.experimental.pallas import tpu_sc as plsc

NC = 2
NS = 16
NW = NC * NS

D_MODEL = 2048
B_TOTAL = 4 * 4096
B_PER_W = B_TOTAL // NW
CHUNK = 16
N_CHUNKS = B_PER_W // CHUNK


def _make_gather():
  mesh = plsc.VectorSubcoreMesh(
      core_axis_name="c", subcore_axis_name="s",
      num_cores=NC, num_subcores=NS)

  @functools.partial(
      pl.kernel,
      out_type=jax.ShapeDtypeStruct((NW, N_CHUNKS, CHUNK, D_MODEL),
                                    jnp.float32),
      mesh=mesh,
      scratch_types=[
          pltpu.VMEM((N_CHUNKS, CHUNK), jnp.int32),
          pltpu.VMEM((CHUNK, D_MODEL), jnp.float32),
          pltpu.VMEM((CHUNK, D_MODEL), jnp.float32),
          pltpu.SemaphoreType.DMA,
          pltpu.SemaphoreType.DMA,
      ],
  )
  def gather_kernel(idx_hbm, table_hbm, out_hbm, idx_v, buf0, buf1,
                    sem0, sem1):
    wid = lax.axis_index("s") * NC + lax.axis_index("c")
    pltpu.sync_copy(idx_hbm.at[wid], idx_v)

    bufs = (buf0, buf1)
    sems = (sem0, sem1)

    for b in range(2):
      pltpu.async_copy(table_hbm.at[idx_v.at[b]], bufs[b], sems[b])

    @pl.loop(0, N_CHUNKS, step=2)
    def _(j):
      for b in range(2):
        c = j + b
        pltpu.make_async_copy(table_hbm.at[idx_v.at[c]], bufs[b],
                              sems[b]).wait()
        pltpu.sync_copy(bufs[b], out_hbm.at[wid, c])

        @pl.when(c + 2 < N_CHUNKS)
        def _():
          pltpu.async_copy(table_hbm.at[idx_v.at[c + 2]], bufs[b], sems[b])

  return gather_kernel


_gather = _make_gather()


@jax.jit
def kernel(tokens, W_E):
  idx = tokens.reshape(NW, N_CHUNKS, CHUNK).astype(jnp.int32)
  out = _gather(idx, W_E)
  return out.reshape(tokens.shape[0], tokens.shape[1], D_MODEL)
